# baseline (speedup 1.0000x reference)
import functools
import math

import jax
import jax.numpy as jnp
from jax import lax
from jax.experimental import pallas as pl
from jax.experimental.pallas import tpu as pltpu
from jax.experimental.pallas import tpu_sc as plsc

RC = 3.0
B = 1.0
A = 1.0
N_ATOMS = 100000
N_EDGES = 6400000

NC = 2
NS = 16
L = 16
NW = NC * NS
CHUNK = 2048
TOTCH = N_EDGES // CHUNK
IPC = CHUNK // L
U = 32
AR = 896
AC = 128
RPT = AR // NS


@functools.partial(
    pl.kernel,
    out_type=jax.ShapeDtypeStruct((NC, AR, AC), jnp.float32),
    mesh=plsc.VectorSubcoreMesh(
        core_axis_name="c", subcore_axis_name="s", num_cores=NC,
        num_subcores=NS,
    ),
    scratch_types=[
        pltpu.VMEM((AR, AC), jnp.float32),
        pltpu.VMEM((CHUNK,), jnp.float32),
        pltpu.VMEM((CHUNK,), jnp.float32),
        pltpu.VMEM((CHUNK,), jnp.int32),
        pltpu.VMEM((CHUNK,), jnp.int32),
        pltpu.VMEM((AR,), jnp.int32),
        pltpu.VMEM_SHARED((AR, AC), jnp.float32),
        pltpu.SemaphoreType.DMA,
        pltpu.SemaphoreType.DMA,
        pltpu.SemaphoreType.DMA,
        pltpu.SemaphoreType.DMA,
    ],
    compiler_params=pltpu.CompilerParams(needs_layout_passes=False),
)
def _sc_segsum(dist_hbm, ind_hbm, out_hbm, acc, dbuf0, dbuf1, ibuf0, ibuf1,
               rowidx, stage, sd0, sd1, si0, si1):
    cid = lax.axis_index("c")
    sid = lax.axis_index("s")
    wid = sid * NC + cid
    n_w = TOTCH // NW + jnp.where(wid < TOTCH % NW, 1, 0)
    io = lax.iota(jnp.int32, L)
    zero = jnp.zeros((L,), jnp.float32)
    zcol = jnp.zeros((L,), jnp.int32)
    e0 = jnp.float32(A * math.exp(-RC / B))
    a_const = jnp.float32(A)

    dbufs = (dbuf0, dbuf1)
    ibufs = (ibuf0, ibuf1)
    dsems = (sd0, sd1)
    isems = (si0, si1)

    def issue(k, buf):
        eb = (wid + NW * k) * CHUNK
        pltpu.async_copy(dist_hbm.at[pl.ds(eb, CHUNK)], dbufs[buf], dsems[buf])
        pltpu.async_copy(ind_hbm.at[0, pl.ds(eb, CHUNK)], ibufs[buf],
                         isems[buf])

    def wait(buf):
        pltpu.make_async_copy(dist_hbm.at[pl.ds(0, CHUNK)], dbufs[buf],
                              dsems[buf]).wait()
        pltpu.make_async_copy(ind_hbm.at[0, pl.ds(0, CHUNK)], ibufs[buf],
                              isems[buf]).wait()

    def process(buf):
        @plsc.parallel_loop(0, IPC, 1, unroll=U)
        def body(i):
            b16 = i * L
            d = dbufs[buf][pl.ds(b16, L)]
            idxv = ibufs[buf][pl.ds(b16, L)]
            env = a_const * jnp.exp(-d) - e0
            plsc.addupdate_scatter(acc, [zcol, idxv], env)

    @pl.when(n_w > 0)
    def _():
        issue(0, 0)

    @pl.when(n_w > 1)
    def _():
        issue(1, 1)

    def zinit(j, carry):
        for k in range(AC // L):
            acc[j, pl.ds(k * L, L)] = zero
        return carry
    lax.fori_loop(0, AR, zinit, 0)

    def iinit(j, carry):
        rowidx[pl.ds(j * L, L)] = io + j * L
        return carry
    lax.fori_loop(0, AR // L, iinit, 0)

    pltpu.sync_copy(acc.at[pl.ds(0, RPT)], stage.at[pl.ds(sid * RPT, RPT)])

    def outer(j, carry):
        k0 = 2 * j
        k1 = 2 * j + 1

        @pl.when(k0 < n_w)
        def _():
            wait(0)
            process(0)

        @pl.when(k0 + 2 < n_w)
        def _():
            issue(k0 + 2, 0)

        @pl.when(k1 < n_w)
        def _():
            wait(1)
            process(1)

        @pl.when(k1 + 2 < n_w)
        def _():
            issue(k1 + 2, 1)

        return carry
    lax.fori_loop(0, (n_w + 1) // 2, outer, 0)

    plsc.subcore_barrier()
    pltpu.sync_copy(acc, stage.at[rowidx], add=True)
    plsc.subcore_barrier()
    pltpu.sync_copy(stage.at[pl.ds(sid * RPT, RPT)],
                    out_hbm.at[cid, pl.ds(sid * RPT, RPT)])


def _tc_combine(p_ref, o_ref):
    s = (p_ref[0] + p_ref[1]) * 0.5
    o_ref[...] = s.reshape(-1)[:N_ATOMS]


_combine = pl.pallas_call(
    _tc_combine,
    out_shape=jax.ShapeDtypeStruct((N_ATOMS,), jnp.float32),
)


def kernel(dist, ind_1, ind_2):
    del ind_1
    idt = ind_2.astype(jnp.int32).T
    partials = _sc_segsum(dist, idt)
    return _combine(partials)


# speedup vs baseline: 1.0064x; 1.0064x over previous
import functools
import math

import jax
import jax.numpy as jnp
from jax import lax
from jax.experimental import pallas as pl
from jax.experimental.pallas import tpu as pltpu
from jax.experimental.pallas import tpu_sc as plsc

RC = 3.0
B = 1.0
A = 1.0
N_ATOMS = 100000
N_EDGES = 6400000

NC = 2
NS = 16
L = 16
NW = NC * NS
CHUNK = 2048
TOTCH = N_EDGES // CHUNK
IPC = CHUNK // L
U = 16
AR = 896
AC = 128
RPT = AR // NS


@functools.partial(
    pl.kernel,
    out_type=jax.ShapeDtypeStruct((NC, AR, AC), jnp.float32),
    mesh=plsc.VectorSubcoreMesh(
        core_axis_name="c", subcore_axis_name="s", num_cores=NC,
        num_subcores=NS,
    ),
    scratch_types=[
        pltpu.VMEM((AR, AC), jnp.float32),
        pltpu.VMEM((CHUNK,), jnp.float32),
        pltpu.VMEM((CHUNK,), jnp.float32),
        pltpu.VMEM((CHUNK,), jnp.int32),
        pltpu.VMEM((CHUNK,), jnp.int32),
        pltpu.VMEM((AR,), jnp.int32),
        pltpu.VMEM_SHARED((AR, AC), jnp.float32),
        pltpu.SemaphoreType.DMA,
        pltpu.SemaphoreType.DMA,
        pltpu.SemaphoreType.DMA,
        pltpu.SemaphoreType.DMA,
    ],
    compiler_params=pltpu.CompilerParams(needs_layout_passes=False),
)
def _sc_segsum(dist_hbm, ind_hbm, out_hbm, acc, dbuf0, dbuf1, ibuf0, ibuf1,
               rowidx, stage, sd0, sd1, si0, si1):
    cid = lax.axis_index("c")
    sid = lax.axis_index("s")
    wid = sid * NC + cid
    n_w = TOTCH // NW + jnp.where(wid < TOTCH % NW, 1, 0)
    io = lax.iota(jnp.int32, L)
    zero = jnp.zeros((L,), jnp.float32)
    zcol = jnp.zeros((L,), jnp.int32)
    e0 = jnp.float32(A * math.exp(-RC / B))
    a_const = jnp.float32(A)

    dbufs = (dbuf0, dbuf1)
    ibufs = (ibuf0, ibuf1)
    dsems = (sd0, sd1)
    isems = (si0, si1)

    def issue(k, buf):
        eb = (wid + NW * k) * CHUNK
        pltpu.async_copy(dist_hbm.at[pl.ds(eb, CHUNK)], dbufs[buf], dsems[buf])
        pltpu.async_copy(ind_hbm.at[0, pl.ds(eb, CHUNK)], ibufs[buf],
                         isems[buf])

    def wait(buf):
        pltpu.make_async_copy(dist_hbm.at[pl.ds(0, CHUNK)], dbufs[buf],
                              dsems[buf]).wait()
        pltpu.make_async_copy(ind_hbm.at[0, pl.ds(0, CHUNK)], ibufs[buf],
                              isems[buf]).wait()

    def process(buf):
        @plsc.parallel_loop(0, IPC, 1, unroll=U)
        def body(i):
            b16 = i * L
            d = dbufs[buf][pl.ds(b16, L)]
            idxv = ibufs[buf][pl.ds(b16, L)]
            env = a_const * jnp.exp(-d) - e0
            plsc.addupdate_scatter(acc, [zcol, idxv], env)

    @pl.when(n_w > 0)
    def _():
        issue(0, 0)

    @pl.when(n_w > 1)
    def _():
        issue(1, 1)

    def zinit(j, carry):
        for k in range(AC // L):
            acc[j, pl.ds(k * L, L)] = zero
        return carry
    lax.fori_loop(0, AR, zinit, 0)

    def iinit(j, carry):
        rowidx[pl.ds(j * L, L)] = io + j * L
        return carry
    lax.fori_loop(0, AR // L, iinit, 0)

    pltpu.sync_copy(acc.at[pl.ds(0, RPT)], stage.at[pl.ds(sid * RPT, RPT)])

    def outer(j, carry):
        k0 = 2 * j
        k1 = 2 * j + 1

        @pl.when(k0 < n_w)
        def _():
            wait(0)
            process(0)

        @pl.when(k0 + 2 < n_w)
        def _():
            issue(k0 + 2, 0)

        @pl.when(k1 < n_w)
        def _():
            wait(1)
            process(1)

        @pl.when(k1 + 2 < n_w)
        def _():
            issue(k1 + 2, 1)

        return carry
    lax.fori_loop(0, (n_w + 1) // 2, outer, 0)

    plsc.subcore_barrier()
    pltpu.sync_copy(acc, stage.at[rowidx], add=True)
    plsc.subcore_barrier()
    pltpu.sync_copy(stage.at[pl.ds(sid * RPT, RPT)],
                    out_hbm.at[cid, pl.ds(sid * RPT, RPT)])


def _tc_combine(p_ref, o_ref):
    s = (p_ref[0] + p_ref[1]) * 0.5
    o_ref[...] = s.reshape(-1)[:N_ATOMS]


_combine = pl.pallas_call(
    _tc_combine,
    out_shape=jax.ShapeDtypeStruct((N_ATOMS,), jnp.float32),
)


def kernel(dist, ind_1, ind_2):
    del ind_1
    idt = ind_2.astype(jnp.int32).T
    partials = _sc_segsum(dist, idt)
    return _combine(partials)
